# baseline (speedup 1.0000x reference)
import functools

import jax
import jax.numpy as jnp
from jax import lax
from jax.experimental import pallas as pl
from jax.experimental.pallas import tpu as pltpu
from jax.experimental.pallas import tpu_sc as plsc

BATCH = 1024
NUM_CLASSES = 100000

NC = 2
NS = 16
L = 16
NW = NC * NS
RPW = BATCH // NW
G = 8
NG = RPW // G
CC = 3968
NFULL = 25
MINI = 768
TAILW = 32
_INT_MAX = 0x7FFFFFFF


def _scan_rows(buf, maxacc, idxacc, colbase, cols):
  steps = cols // (2 * L)
  for rr in range(G):
    a0 = maxacc[rr, pl.ds(0, L)]
    a1 = maxacc[rr, pl.ds(L, L)]
    i0 = idxacc[rr, pl.ds(0, L)]
    i1 = idxacc[rr, pl.ds(L, L)]
    vi0 = lax.iota(jnp.int32, L) + colbase
    vi1 = vi0 + L

    def step(i, st, rr=rr):
      a0, i0, a1, i1, vi0, vi1 = st
      off = i * (2 * L)
      v0 = buf[rr, pl.ds(off, L)]
      v1 = buf[rr, pl.ds(off + L, L)]
      p0 = v0 > a0
      p1 = v1 > a1
      a0 = jnp.where(p0, v0, a0)
      i0 = jnp.where(p0, vi0, i0)
      a1 = jnp.where(p1, v1, a1)
      i1 = jnp.where(p1, vi1, i1)
      return (a0, i0, a1, i1, vi0 + 2 * L, vi1 + 2 * L)

    a0, i0, a1, i1, _, _ = lax.fori_loop(
        0, steps, step, (a0, i0, a1, i1, vi0, vi1))
    maxacc[rr, pl.ds(0, L)] = a0
    maxacc[rr, pl.ds(L, L)] = a1
    idxacc[rr, pl.ds(0, L)] = i0
    idxacc[rr, pl.ds(L, L)] = i1


def _argmax_body(x_hbm, xtail_hbm, lab_hbm, out_hbm, buf0, buf1, tailbuf,
                 maxacc, idxacc, pred_v, lab_v, sem0, sem1):
  wid = lax.axis_index("s") * NC + lax.axis_index("c")
  base = wid * RPW

  def start(c, buf, sem, row8):
    pltpu.make_async_copy(
        x_hbm.at[pl.ds(row8, G), pl.ds(c * CC, CC)], buf, sem).start()

  def group_body(g, pred_acc):
    row8 = base + g * G
    neg_inf = jnp.full((L,), -jnp.inf, jnp.float32)
    zero = jnp.zeros((L,), jnp.int32)
    for rr in range(G):
      maxacc[rr, pl.ds(0, L)] = neg_inf
      maxacc[rr, pl.ds(L, L)] = neg_inf
      idxacc[rr, pl.ds(0, L)] = zero
      idxacc[rr, pl.ds(L, L)] = zero

    start(0, buf0, sem0, row8)

    def chunk_pair(c2, carry):
      c = c2 * 2
      pltpu.make_async_copy(x_hbm.at[pl.ds(row8, G), pl.ds(c * CC, CC)],
                            buf0, sem0).wait()
      start(c + 1, buf1, sem1, row8)
      _scan_rows(buf0, maxacc, idxacc, c * CC, CC)
      pltpu.make_async_copy(x_hbm.at[pl.ds(row8, G), pl.ds((c + 1) * CC, CC)],
                            buf1, sem1).wait()
      start(c + 2, buf0, sem0, row8)
      _scan_rows(buf1, maxacc, idxacc, (c + 1) * CC, CC)
      return carry

    lax.fori_loop(0, (NFULL - 1) // 2, chunk_pair, 0)
    last = NFULL - 1
    pltpu.make_async_copy(x_hbm.at[pl.ds(row8, G), pl.ds(last * CC, CC)],
                          buf0, sem0).wait()
    pltpu.make_async_copy(
        x_hbm.at[pl.ds(row8, G), pl.ds(NFULL * CC, MINI)],
        buf1.at[:, pl.ds(0, MINI)], sem1).start()
    _scan_rows(buf0, maxacc, idxacc, last * CC, CC)
    pltpu.make_async_copy(
        x_hbm.at[pl.ds(row8, G), pl.ds(NFULL * CC, MINI)],
        buf1.at[:, pl.ds(0, MINI)], sem1).wait()
    pltpu.make_async_copy(xtail_hbm.at[pl.ds(row8, G)], tailbuf, sem0).start()
    _scan_rows(buf1, maxacc, idxacc, NFULL * CC, MINI)
    pltpu.make_async_copy(xtail_hbm.at[pl.ds(row8, G)], tailbuf, sem0).wait()
    _scan_rows(tailbuf, maxacc, idxacc, NFULL * CC + MINI, TAILW)

    pv0, pv1 = pred_acc
    lane_iota = lax.iota(jnp.int32, L)
    big = jnp.full((L,), _INT_MAX, jnp.int32)
    for rr in range(G):
      a0 = maxacc[rr, pl.ds(0, L)]
      a1 = maxacc[rr, pl.ds(L, L)]
      i0 = idxacc[rr, pl.ds(0, L)]
      i1 = idxacc[rr, pl.ds(L, L)]
      take1 = (a1 > a0) | ((a1 == a0) & (i1 < i0))
      vmax = jnp.where(take1, a1, a0)
      vidx = jnp.where(take1, i1, i0)
      m = lax.reduce_max(vmax, (0,))
      cand = jnp.where(vmax == m, vidx, big)
      idx = lax.reduce_min(cand, (0,))
      r = g * G + rr
      lanesel = lane_iota == (r & (L - 1))
      in0 = r < L
      pv0 = jnp.where(lanesel & in0, idx, pv0)
      pv1 = jnp.where(lanesel & (~in0), idx, pv1)
    return (pv0, pv1)

  pv0 = jnp.zeros((L,), jnp.int32)
  pv1 = jnp.zeros((L,), jnp.int32)
  pv0, pv1 = lax.fori_loop(0, NG, group_body, (pv0, pv1))
  pred_v[pl.ds(0, L)] = pv0
  pred_v[pl.ds(L, L)] = pv1

  pltpu.async_copy(lab_hbm.at[pred_v], lab_v, sem0).wait()
  pltpu.sync_copy(lab_v, out_hbm.at[pl.ds(base, RPW)])


@jax.jit
def _run(inputs, inputs_tail, labels_i32):
  mesh = plsc.VectorSubcoreMesh(core_axis_name="c", subcore_axis_name="s")
  f = functools.partial(
      pl.kernel,
      out_type=jax.ShapeDtypeStruct((BATCH,), jnp.int32),
      mesh=mesh,
      compiler_params=pltpu.CompilerParams(needs_layout_passes=False),
      scratch_types=[
          pltpu.VMEM((G, CC), jnp.float32),
          pltpu.VMEM((G, CC), jnp.float32),
          pltpu.VMEM((G, TAILW), jnp.float32),
          pltpu.VMEM((G, 2 * L), jnp.float32),
          pltpu.VMEM((G, 2 * L), jnp.int32),
          pltpu.VMEM((RPW,), jnp.int32),
          pltpu.VMEM((RPW,), jnp.int32),
          pltpu.SemaphoreType.DMA,
          pltpu.SemaphoreType.DMA,
      ],
  )(_argmax_body)
  return f(inputs, inputs_tail, labels_i32)


def kernel(inputs, labels):
  inputs_tail = inputs[:, NUM_CLASSES - TAILW:]
  out = _run(inputs, inputs_tail, labels.astype(jnp.int32))
  return out.astype(labels.dtype)


# speedup vs baseline: 1.2567x; 1.2567x over previous
import functools

import jax
import jax.numpy as jnp
from jax import lax
from jax.experimental import pallas as pl
from jax.experimental.pallas import tpu as pltpu
from jax.experimental.pallas import tpu_sc as plsc

BATCH = 1024
NUM_CLASSES = 100000

NC = 2
NS = 16
L = 16
NW = NC * NS
RPW = BATCH // NW
G = 8
NG = RPW // G
CC = 3968
NFULL = 25
MINI = 768
TAILW = 32
_INT_MAX = 0x7FFFFFFF


NPAIR = 4
UNIT = 8 * L


def _upd(accs, idxs, v, vik, p):
  m = v > accs[p]
  accs[p] = jnp.maximum(accs[p], v)
  idxs[p] = jnp.where(m, vik, idxs[p])


def _scan_rows(buf, maxacc, idxacc, colbase, cols):
  full = cols // UNIT
  rem = (cols - full * UNIT) // L
  for rr in range(G):
    accs = [maxacc[rr, pl.ds(p * L, L)] for p in range(NPAIR)]
    idxs = [idxacc[rr, pl.ds(p * L, L)] for p in range(NPAIR)]
    vi0 = lax.iota(jnp.int32, L) + colbase

    def step(s, st, rr=rr):
      st = list(st)
      vi = st[-1]
      accs = st[0:NPAIR]
      idxs = st[NPAIR:2 * NPAIR]
      off = s * UNIT
      for k in range(8):
        v = buf[rr, pl.ds(off + k * L, L)]
        _upd(accs, idxs, v, vi + k * L, k & (NPAIR - 1))
      return (*accs, *idxs, vi + UNIT)

    st = lax.fori_loop(0, full, step, (*accs, *idxs, vi0))
    accs = list(st[0:NPAIR])
    idxs = list(st[NPAIR:2 * NPAIR])
    vi = st[-1]
    for k in range(rem):
      v = buf[rr, pl.ds(full * UNIT + k * L, L)]
      _upd(accs, idxs, v, vi + k * L, k & (NPAIR - 1))
    for p in range(NPAIR):
      maxacc[rr, pl.ds(p * L, L)] = accs[p]
      idxacc[rr, pl.ds(p * L, L)] = idxs[p]


def _argmax_body(x_hbm, xtail_hbm, lab_hbm, out_hbm, buf0, buf1, tailbuf,
                 maxacc, idxacc, pred_v, lab_v, sem0, sem1):
  wid = lax.axis_index("s") * NC + lax.axis_index("c")
  base = wid * RPW

  def start(c, buf, sem, row8):
    pltpu.make_async_copy(
        x_hbm.at[pl.ds(row8, G), pl.ds(c * CC, CC)], buf, sem).start()

  def group_body(g, pred_acc):
    row8 = base + g * G
    neg_inf = jnp.full((L,), -jnp.inf, jnp.float32)
    zero = jnp.zeros((L,), jnp.int32)
    for rr in range(G):
      for p in range(NPAIR):
        maxacc[rr, pl.ds(p * L, L)] = neg_inf
        idxacc[rr, pl.ds(p * L, L)] = zero

    start(0, buf0, sem0, row8)

    def chunk_pair(c2, carry):
      c = c2 * 2
      pltpu.make_async_copy(x_hbm.at[pl.ds(row8, G), pl.ds(c * CC, CC)],
                            buf0, sem0).wait()
      start(c + 1, buf1, sem1, row8)
      _scan_rows(buf0, maxacc, idxacc, c * CC, CC)
      pltpu.make_async_copy(x_hbm.at[pl.ds(row8, G), pl.ds((c + 1) * CC, CC)],
                            buf1, sem1).wait()
      start(c + 2, buf0, sem0, row8)
      _scan_rows(buf1, maxacc, idxacc, (c + 1) * CC, CC)
      return carry

    lax.fori_loop(0, (NFULL - 1) // 2, chunk_pair, 0)
    last = NFULL - 1
    pltpu.make_async_copy(x_hbm.at[pl.ds(row8, G), pl.ds(last * CC, CC)],
                          buf0, sem0).wait()
    pltpu.make_async_copy(
        x_hbm.at[pl.ds(row8, G), pl.ds(NFULL * CC, MINI)],
        buf1.at[:, pl.ds(0, MINI)], sem1).start()
    _scan_rows(buf0, maxacc, idxacc, last * CC, CC)
    pltpu.make_async_copy(
        x_hbm.at[pl.ds(row8, G), pl.ds(NFULL * CC, MINI)],
        buf1.at[:, pl.ds(0, MINI)], sem1).wait()
    pltpu.make_async_copy(xtail_hbm.at[pl.ds(row8, G)], tailbuf, sem0).start()
    _scan_rows(buf1, maxacc, idxacc, NFULL * CC, MINI)
    pltpu.make_async_copy(xtail_hbm.at[pl.ds(row8, G)], tailbuf, sem0).wait()
    _scan_rows(tailbuf, maxacc, idxacc, NFULL * CC + MINI, TAILW)

    pv0, pv1 = pred_acc
    lane_iota = lax.iota(jnp.int32, L)
    big = jnp.full((L,), _INT_MAX, jnp.int32)
    def merge(a, i, b, j):
      t = (b > a) | ((b == a) & (j < i))
      return jnp.where(t, b, a), jnp.where(t, j, i)

    for rr in range(G):
      accs = [maxacc[rr, pl.ds(p * L, L)] for p in range(NPAIR)]
      idxs = [idxacc[rr, pl.ds(p * L, L)] for p in range(NPAIR)]
      a01, i01 = merge(accs[0], idxs[0], accs[1], idxs[1])
      a23, i23 = merge(accs[2], idxs[2], accs[3], idxs[3])
      vmax, vidx = merge(a01, i01, a23, i23)
      m = lax.reduce_max(vmax, (0,))
      cand = jnp.where(vmax == m, vidx, big)
      idx = lax.reduce_min(cand, (0,))
      r = g * G + rr
      lanesel = lane_iota == (r & (L - 1))
      in0 = r < L
      pv0 = jnp.where(lanesel & in0, idx, pv0)
      pv1 = jnp.where(lanesel & (~in0), idx, pv1)
    return (pv0, pv1)

  pv0 = jnp.zeros((L,), jnp.int32)
  pv1 = jnp.zeros((L,), jnp.int32)
  pv0, pv1 = lax.fori_loop(0, NG, group_body, (pv0, pv1))
  pred_v[pl.ds(0, L)] = pv0
  pred_v[pl.ds(L, L)] = pv1

  pltpu.async_copy(lab_hbm.at[pred_v], lab_v, sem0).wait()
  pltpu.sync_copy(lab_v, out_hbm.at[pl.ds(base, RPW)])


@jax.jit
def _run(inputs, inputs_tail, labels_i32):
  mesh = plsc.VectorSubcoreMesh(core_axis_name="c", subcore_axis_name="s")
  f = functools.partial(
      pl.kernel,
      out_type=jax.ShapeDtypeStruct((BATCH,), jnp.int32),
      mesh=mesh,
      compiler_params=pltpu.CompilerParams(needs_layout_passes=False),
      scratch_types=[
          pltpu.VMEM((G, CC), jnp.float32),
          pltpu.VMEM((G, CC), jnp.float32),
          pltpu.VMEM((G, TAILW), jnp.float32),
          pltpu.VMEM((G, NPAIR * L), jnp.float32),
          pltpu.VMEM((G, NPAIR * L), jnp.int32),
          pltpu.VMEM((RPW,), jnp.int32),
          pltpu.VMEM((RPW,), jnp.int32),
          pltpu.SemaphoreType.DMA,
          pltpu.SemaphoreType.DMA,
      ],
  )(_argmax_body)
  return f(inputs, inputs_tail, labels_i32)


def kernel(inputs, labels):
  inputs_tail = inputs[:, NUM_CLASSES - TAILW:]
  out = _run(inputs, inputs_tail, labels.astype(jnp.int32))
  return out.astype(labels.dtype)


# speedup vs baseline: 1.2604x; 1.0030x over previous
import functools

import jax
import jax.numpy as jnp
from jax import lax
from jax.experimental import pallas as pl
from jax.experimental.pallas import tpu as pltpu
from jax.experimental.pallas import tpu_sc as plsc

BATCH = 1024
NUM_CLASSES = 100000

NC = 2
NS = 16
L = 16
NW = NC * NS
RPW = BATCH // NW
G = 8
NG = RPW // G
CC = 3968
NFULL = 25
MINI = 768
TAILW = 32
_INT_MAX = 0x7FFFFFFF


NPAIR = 8
UNIT = 8 * L


def _scan_rows(buf, maxacc, idxacc, colbase, cols):
  full = cols // UNIT
  rem = (cols - full * UNIT) // L
  step0 = colbase // UNIT
  for rr in range(G):
    accs = tuple(maxacc[rr, pl.ds(p * L, L)] for p in range(NPAIR))
    idxs = tuple(idxacc[rr, pl.ds(p * L, L)] for p in range(NPAIR))
    vstep = jnp.full((L,), step0, jnp.int32)

    def body(off, st, rr=rr):
      accs, idxs, vstep = st
      accs, idxs = list(accs), list(idxs)
      for k in range(NPAIR):
        v = buf[rr, pl.ds(off + k * L, L)]
        m = v > accs[k]
        accs[k] = jnp.maximum(accs[k], v)
        idxs[k] = jnp.where(m, vstep, idxs[k])
      return (tuple(accs), tuple(idxs), vstep + 1)

    if full > 0:
      accs, idxs, vstep = plsc.parallel_loop(
          0, full * UNIT, step=UNIT, carry=(accs, idxs, vstep))(body)
    accs, idxs = list(accs), list(idxs)
    for k in range(rem):
      v = buf[rr, pl.ds(full * UNIT + k * L, L)]
      m = v > accs[k]
      accs[k] = jnp.maximum(accs[k], v)
      idxs[k] = jnp.where(m, vstep, idxs[k])
    for p in range(NPAIR):
      maxacc[rr, pl.ds(p * L, L)] = accs[p]
      idxacc[rr, pl.ds(p * L, L)] = idxs[p]


def _argmax_body(x_hbm, xtail_hbm, lab_hbm, out_hbm, buf0, buf1, tailbuf,
                 maxacc, idxacc, pred_v, lab_v, sem0, sem1):
  wid = lax.axis_index("s") * NC + lax.axis_index("c")
  base = wid * RPW

  def start(c, buf, sem, row8):
    pltpu.make_async_copy(
        x_hbm.at[pl.ds(row8, G), pl.ds(c * CC, CC)], buf, sem).start()

  def group_body(g, pred_acc):
    row8 = base + g * G
    neg_inf = jnp.full((L,), -jnp.inf, jnp.float32)
    zero = jnp.zeros((L,), jnp.int32)
    for rr in range(G):
      for p in range(NPAIR):
        maxacc[rr, pl.ds(p * L, L)] = neg_inf
        idxacc[rr, pl.ds(p * L, L)] = zero

    start(0, buf0, sem0, row8)

    def chunk_pair(c2, carry):
      c = c2 * 2
      pltpu.make_async_copy(x_hbm.at[pl.ds(row8, G), pl.ds(c * CC, CC)],
                            buf0, sem0).wait()
      start(c + 1, buf1, sem1, row8)
      _scan_rows(buf0, maxacc, idxacc, c * CC, CC)
      pltpu.make_async_copy(x_hbm.at[pl.ds(row8, G), pl.ds((c + 1) * CC, CC)],
                            buf1, sem1).wait()
      start(c + 2, buf0, sem0, row8)
      _scan_rows(buf1, maxacc, idxacc, (c + 1) * CC, CC)
      return carry

    lax.fori_loop(0, (NFULL - 1) // 2, chunk_pair, 0)
    last = NFULL - 1
    pltpu.make_async_copy(x_hbm.at[pl.ds(row8, G), pl.ds(last * CC, CC)],
                          buf0, sem0).wait()
    pltpu.make_async_copy(
        x_hbm.at[pl.ds(row8, G), pl.ds(NFULL * CC, MINI)],
        buf1.at[:, pl.ds(0, MINI)], sem1).start()
    _scan_rows(buf0, maxacc, idxacc, last * CC, CC)
    pltpu.make_async_copy(
        x_hbm.at[pl.ds(row8, G), pl.ds(NFULL * CC, MINI)],
        buf1.at[:, pl.ds(0, MINI)], sem1).wait()
    pltpu.make_async_copy(xtail_hbm.at[pl.ds(row8, G)], tailbuf, sem0).start()
    _scan_rows(buf1, maxacc, idxacc, NFULL * CC, MINI)
    pltpu.make_async_copy(xtail_hbm.at[pl.ds(row8, G)], tailbuf, sem0).wait()
    _scan_rows(tailbuf, maxacc, idxacc, NFULL * CC + MINI, TAILW)

    pv0, pv1 = pred_acc
    lane_iota = lax.iota(jnp.int32, L)
    big = jnp.full((L,), _INT_MAX, jnp.int32)
    def merge(a, i, b, j):
      t = (b > a) | ((b == a) & (j < i))
      return jnp.where(t, b, a), jnp.where(t, j, i)

    for rr in range(G):
      accs = [maxacc[rr, pl.ds(p * L, L)] for p in range(NPAIR)]
      idxs = [idxacc[rr, pl.ds(p * L, L)] * UNIT + (lane_iota + p * L)
              for p in range(NPAIR)]
      while len(accs) > 1:
        na, ni = [], []
        for q in range(0, len(accs), 2):
          a, i = merge(accs[q], idxs[q], accs[q + 1], idxs[q + 1])
          na.append(a)
          ni.append(i)
        accs, idxs = na, ni
      vmax, vidx = accs[0], idxs[0]
      m = lax.reduce_max(vmax, (0,))
      cand = jnp.where(vmax == m, vidx, big)
      idx = lax.reduce_min(cand, (0,))
      r = g * G + rr
      lanesel = lane_iota == (r & (L - 1))
      in0 = r < L
      pv0 = jnp.where(lanesel & in0, idx, pv0)
      pv1 = jnp.where(lanesel & (~in0), idx, pv1)
    return (pv0, pv1)

  pv0 = jnp.zeros((L,), jnp.int32)
  pv1 = jnp.zeros((L,), jnp.int32)
  pv0, pv1 = lax.fori_loop(0, NG, group_body, (pv0, pv1))
  pred_v[pl.ds(0, L)] = pv0
  pred_v[pl.ds(L, L)] = pv1

  pltpu.async_copy(lab_hbm.at[pred_v], lab_v, sem0).wait()
  pltpu.sync_copy(lab_v, out_hbm.at[pl.ds(base, RPW)])


@jax.jit
def _run(inputs, inputs_tail, labels_i32):
  mesh = plsc.VectorSubcoreMesh(core_axis_name="c", subcore_axis_name="s")
  f = functools.partial(
      pl.kernel,
      out_type=jax.ShapeDtypeStruct((BATCH,), jnp.int32),
      mesh=mesh,
      compiler_params=pltpu.CompilerParams(needs_layout_passes=False),
      scratch_types=[
          pltpu.VMEM((G, CC), jnp.float32),
          pltpu.VMEM((G, CC), jnp.float32),
          pltpu.VMEM((G, TAILW), jnp.float32),
          pltpu.VMEM((G, NPAIR * L), jnp.float32),
          pltpu.VMEM((G, NPAIR * L), jnp.int32),
          pltpu.VMEM((RPW,), jnp.int32),
          pltpu.VMEM((RPW,), jnp.int32),
          pltpu.SemaphoreType.DMA,
          pltpu.SemaphoreType.DMA,
      ],
  )(_argmax_body)
  return f(inputs, inputs_tail, labels_i32)


def kernel(inputs, labels):
  inputs_tail = inputs[:, NUM_CLASSES - TAILW:]
  out = _run(inputs, inputs_tail, labels.astype(jnp.int32))
  return out.astype(labels.dtype)
